# Initial kernel scaffold; baseline (speedup 1.0000x reference)
import functools

import jax
import jax.numpy as jnp
from jax import lax
from jax.experimental import pallas as pl
from jax.experimental.pallas import tpu as pltpu
from jax.experimental.pallas import tpu_sc as plsc

BATCH = 4096
FIELD_COUNT = 26
DIM = 16
NNZ = 212992
N_SEG = BATCH * FIELD_COUNT

NUM_CORES = 2
NUM_SUBCORES = 16
NUM_LANES = 16

BLK = 128
NB_TOT = NNZ // BLK
NBMAX = -(-NB_TOT // NUM_SUBCORES)
NB_PAD = NB_TOT + NBMAX
PAD_LEN = NB_PAD * BLK

HALF_SEG = N_SEG // NUM_CORES
ROWS_PER_SUB = HALF_SEG // NUM_SUBCORES

_mesh = plsc.VectorSubcoreMesh(core_axis_name="c", subcore_axis_name="s")


@functools.partial(
    pl.kernel,
    out_type=jax.ShapeDtypeStruct((N_SEG, DIM), jnp.float32),
    mesh=_mesh,
    scratch_types=[
        pltpu.SMEM((8,), jnp.int32),
        pltpu.VMEM((NBMAX, BLK), jnp.int32),
        pltpu.VMEM((NBMAX, BLK), jnp.float32),
        pltpu.VMEM((NBMAX, BLK), jnp.int32),
        pltpu.VMEM((BLK, DIM), jnp.float32),
        pltpu.VMEM((1, BLK), jnp.int32),
        pltpu.VMEM_SHARED((HALF_SEG, DIM), jnp.float32),
    ],
)
def _embed_sum(ids_hbm, w_hbm, segs_hbm, emb_hbm, bounds_hbm, out_hbm,
               bounds_sm, ids_v, w_v, segs_v, rows_v, segidx_v, acc_sh):
    c = lax.axis_index("c")
    s = lax.axis_index("s")

    pltpu.sync_copy(bounds_hbm, bounds_sm)
    lo_b = jnp.where(c == 0, 0, bounds_sm[1])
    hi_b = jnp.where(c == 0, bounds_sm[0], NB_TOT)
    nb = hi_b - lo_b
    my_lo = lo_b + (nb * s) // NUM_SUBCORES
    my_hi = lo_b + (nb * (s + 1)) // NUM_SUBCORES
    n_my = my_hi - my_lo

    pltpu.sync_copy(ids_hbm.at[pl.ds(my_lo, NBMAX)], ids_v)
    pltpu.sync_copy(w_hbm.at[pl.ds(my_lo, NBMAX)], w_v)
    pltpu.sync_copy(segs_hbm.at[pl.ds(my_lo, NBMAX)], segs_v)

    zero = jnp.zeros((NUM_LANES,), jnp.float32)

    @pl.loop(0, BLK)
    def _(i):
        rows_v[i, :] = zero

    row0 = s * ROWS_PER_SUB

    @pl.loop(0, ROWS_PER_SUB // BLK)
    def _(k):
        pltpu.sync_copy(rows_v, acc_sh.at[pl.ds(row0 + k * BLK, BLK)])

    plsc.subcore_barrier()

    seg_base = c * HALF_SEG

    @pl.loop(0, n_my)
    def _(j):
        pltpu.sync_copy(emb_hbm.at[ids_v.at[j]], rows_v)
        for g in range(BLK // NUM_LANES):
            sl = pl.ds(g * NUM_LANES, NUM_LANES)
            segv = segs_v[j, sl]
            wv = w_v[j, sl]
            own = (segv >= seg_base) & (segv < seg_base + HALF_SEG)
            wok = jnp.where(own, wv, 0.0)
            segloc = jnp.where(own, segv - seg_base, 0)
            segidx_v[0, sl] = segloc
            ridx = lax.iota(jnp.int32, NUM_LANES) + (g * NUM_LANES)
            for d in range(DIM):
                cidx = jnp.full((NUM_LANES,), d, jnp.int32)
                v = plsc.load_gather(rows_v, [ridx, cidx])
                plsc.store_scatter(rows_v, [ridx, cidx], v * wok)
        pltpu.sync_copy(rows_v, acc_sh.at[segidx_v.at[0]], add=True)

    plsc.subcore_barrier()

    pltpu.sync_copy(acc_sh.at[pl.ds(row0, ROWS_PER_SUB)],
                    out_hbm.at[pl.ds(seg_base + row0, ROWS_PER_SUB)])


def kernel(feat_ids, feat_weights, segment_ids, embedding):
    ids = feat_ids.astype(jnp.int32)
    segs = segment_ids.astype(jnp.int32)
    w = feat_weights.astype(jnp.float32)

    split = jnp.searchsorted(segs, HALF_SEG).astype(jnp.int32)
    bounds = jnp.pad(
        jnp.stack([(split + BLK - 1) // BLK, split // BLK]), (0, 6)
    ).astype(jnp.int32)

    pad = PAD_LEN - NNZ
    ids_p = jnp.pad(ids, (0, pad)).reshape(NB_PAD, BLK)
    w_p = jnp.pad(w, (0, pad)).reshape(NB_PAD, BLK)
    segs_p = jnp.pad(segs, (0, pad)).reshape(NB_PAD, BLK)

    pooled = _embed_sum(ids_p, w_p, segs_p, embedding, bounds)
    return pooled.reshape(BATCH, FIELD_COUNT * DIM)


# speedup vs baseline: 1.4084x; 1.4084x over previous
import dataclasses
import functools

import jax
import jax.numpy as jnp
from jax import lax
from jax.experimental import pallas as pl
from jax.experimental.pallas import tpu as pltpu
from jax.experimental.pallas import tpu_sc as plsc

BATCH = 4096
FIELD_COUNT = 26
DIM = 16
NNZ = 212992
N_SEG = BATCH * FIELD_COUNT

NUM_CORES = 2
NUM_SUBCORES = 16
NUM_LANES = 16

BLK = 128
NB_TOT = NNZ // BLK
NBMAX = -(-NB_TOT // NUM_SUBCORES)
NB_PAD = NB_TOT + NBMAX
PAD_LEN = NB_PAD * BLK

HALF_SEG = N_SEG // NUM_CORES
ROWS_PER_SUB = HALF_SEG // NUM_SUBCORES

_mesh = plsc.VectorSubcoreMesh(core_axis_name="c", subcore_axis_name="s")

_cp = pltpu.CompilerParams(
    needs_layout_passes=False, use_tc_tiling_on_sc=False
)


@functools.partial(
    pl.kernel,
    out_type=jax.ShapeDtypeStruct((N_SEG, DIM), jnp.float32),
    mesh=_mesh,
    scratch_types=[
        pltpu.VMEM((16,), jnp.int32),
        pltpu.VMEM((NBMAX, BLK), jnp.int32),
        pltpu.VMEM((NBMAX, BLK), jnp.float32),
        pltpu.VMEM((NBMAX, BLK), jnp.int32),
        pltpu.VMEM((BLK, DIM), jnp.float32),
        pltpu.VMEM((1, BLK), jnp.int32),
        pltpu.VMEM_SHARED((HALF_SEG, DIM), jnp.float32),
    ],
    compiler_params=_cp,
)
def _embed_sum(ids_hbm, w_hbm, segs_hbm, emb_hbm, bounds_hbm, out_hbm,
               bounds_v, ids_v, w_v, segs_v, rows_v, segidx_v, acc_sh):
    c = lax.axis_index("c")
    s = lax.axis_index("s")

    pltpu.sync_copy(bounds_hbm, bounds_v)
    bvec = bounds_v[...]
    lo_b = jnp.where(c == 0, 0, bvec[1])
    hi_b = jnp.where(c == 0, bvec[0], NB_TOT)
    nb8 = (hi_b - lo_b) // 8
    my_lo = lo_b + 8 * ((nb8 * s) // NUM_SUBCORES)
    my_hi = lo_b + 8 * ((nb8 * (s + 1)) // NUM_SUBCORES)
    n_my = my_hi - my_lo
    my_lo = pl.multiple_of(my_lo, 8)

    pltpu.sync_copy(ids_hbm.at[pl.ds(my_lo, NBMAX)], ids_v)
    pltpu.sync_copy(w_hbm.at[pl.ds(my_lo, NBMAX)], w_v)
    pltpu.sync_copy(segs_hbm.at[pl.ds(my_lo, NBMAX)], segs_v)

    zero = jnp.zeros((NUM_LANES,), jnp.float32)

    @pl.loop(0, BLK)
    def _(i):
        rows_v[i, :] = zero

    row0 = s * ROWS_PER_SUB

    @pl.loop(0, ROWS_PER_SUB // BLK)
    def _(k):
        pltpu.sync_copy(rows_v, acc_sh.at[pl.ds(row0 + k * BLK, BLK)])

    plsc.subcore_barrier()

    seg_base = c * HALF_SEG

    @pl.loop(0, n_my)
    def _(j):
        pltpu.sync_copy(emb_hbm.at[ids_v.at[j]], rows_v)
        for g in range(BLK // NUM_LANES):
            sl = pl.ds(g * NUM_LANES, NUM_LANES)
            segv = segs_v[j, sl]
            wv = w_v[j, sl]
            own = (segv >= seg_base) & (segv < seg_base + HALF_SEG)
            wok = jnp.where(own, wv, 0.0)
            segloc = jnp.where(own, segv - seg_base, 0)
            segidx_v[0, sl] = segloc
            ridx = lax.iota(jnp.int32, NUM_LANES) + (g * NUM_LANES)
            for d in range(DIM):
                cidx = jnp.full((NUM_LANES,), d, jnp.int32)
                v = plsc.load_gather(rows_v, [ridx, cidx])
                plsc.store_scatter(rows_v, [ridx, cidx], v * wok)
        pltpu.sync_copy(rows_v, acc_sh.at[segidx_v.at[0]], add=True)

    plsc.subcore_barrier()

    pltpu.sync_copy(acc_sh.at[pl.ds(row0, ROWS_PER_SUB)],
                    out_hbm.at[pl.ds(seg_base + row0, ROWS_PER_SUB)])


def kernel(feat_ids, feat_weights, segment_ids, embedding):
    ids = feat_ids.astype(jnp.int32)
    segs = segment_ids.astype(jnp.int32)
    w = feat_weights.astype(jnp.float32)

    split = jnp.searchsorted(segs, HALF_SEG).astype(jnp.int32)
    hi0 = -(-(-(-split // BLK)) // 8) * 8
    lo1 = (split // BLK) // 8 * 8
    bounds = jnp.pad(jnp.stack([hi0, lo1]), (0, 14)).astype(jnp.int32)

    pad = PAD_LEN - NNZ
    ids_p = jnp.pad(ids, (0, pad)).reshape(NB_PAD, BLK)
    w_p = jnp.pad(w, (0, pad)).reshape(NB_PAD, BLK)
    segs_p = jnp.pad(segs, (0, pad)).reshape(NB_PAD, BLK)

    pooled = _embed_sum(ids_p, w_p, segs_p, embedding, bounds)
    return pooled.reshape(BATCH, FIELD_COUNT * DIM)


# speedup vs baseline: 1.5219x; 1.0805x over previous
import dataclasses
import functools

import jax
import jax.numpy as jnp
from jax import lax
from jax.experimental import pallas as pl
from jax.experimental.pallas import tpu as pltpu
from jax.experimental.pallas import tpu_sc as plsc

BATCH = 4096
FIELD_COUNT = 26
DIM = 16
NNZ = 212992
N_SEG = BATCH * FIELD_COUNT

NUM_CORES = 2
NUM_SUBCORES = 16
NUM_LANES = 16

NSLOT = 4
BLK = 128
NB_TOT = NNZ // BLK
NBMAX = -(-NB_TOT // NUM_SUBCORES)
NB_PAD = NB_TOT + NBMAX
PAD_LEN = NB_PAD * BLK

HALF_SEG = N_SEG // NUM_CORES
ROWS_PER_SUB = HALF_SEG // NUM_SUBCORES

_mesh = plsc.VectorSubcoreMesh(core_axis_name="c", subcore_axis_name="s")

_cp = pltpu.CompilerParams(
    needs_layout_passes=False, use_tc_tiling_on_sc=False
)


@functools.partial(
    pl.kernel,
    out_type=jax.ShapeDtypeStruct((N_SEG, DIM), jnp.float32),
    mesh=_mesh,
    scratch_types=[
        pltpu.VMEM((16,), jnp.int32),
        pltpu.VMEM((NBMAX, BLK), jnp.int32),
        pltpu.VMEM((NBMAX, BLK), jnp.float32),
        pltpu.VMEM((NBMAX, BLK), jnp.int32),
        pltpu.VMEM((NSLOT, BLK, DIM), jnp.float32),
        pltpu.VMEM((NSLOT, BLK), jnp.int32),
        pltpu.VMEM_SHARED((HALF_SEG, DIM), jnp.float32),
        pltpu.SemaphoreType.DMA((NSLOT,)),
        pltpu.SemaphoreType.DMA((NSLOT,)),
    ],
    compiler_params=_cp,
)
def _embed_sum(ids_hbm, w_hbm, segs_hbm, emb_hbm, bounds_hbm, out_hbm,
               bounds_v, ids_v, w_v, segs_v, rows_v, segidx_v, acc_sh,
               g_sem, s_sem):
    c = lax.axis_index("c")
    s = lax.axis_index("s")

    pltpu.sync_copy(bounds_hbm, bounds_v)
    bvec = bounds_v[...]
    lo_b = jnp.where(c == 0, 0, bvec[1])
    hi_b = jnp.where(c == 0, bvec[0], NB_TOT)
    nb8 = (hi_b - lo_b) // 8
    my_lo = lo_b + 8 * ((nb8 * s) // NUM_SUBCORES)
    my_hi = lo_b + 8 * ((nb8 * (s + 1)) // NUM_SUBCORES)
    n_my = my_hi - my_lo
    my_lo = pl.multiple_of(my_lo, 8)

    pltpu.sync_copy(ids_hbm.at[pl.ds(my_lo, NBMAX)], ids_v)
    pltpu.sync_copy(w_hbm.at[pl.ds(my_lo, NBMAX)], w_v)
    pltpu.sync_copy(segs_hbm.at[pl.ds(my_lo, NBMAX)], segs_v)

    seg_base = c * HALF_SEG

    def g_desc(jj, slot):
        return pltpu.make_async_copy(
            emb_hbm.at[ids_v.at[jj]], rows_v.at[slot], g_sem.at[slot])

    def s_desc(slot):
        return pltpu.make_async_copy(
            rows_v.at[slot], acc_sh.at[segidx_v.at[slot]], s_sem.at[slot])

    def compute(jj, slot):
        for g in range(BLK // NUM_LANES):
            sl = pl.ds(g * NUM_LANES, NUM_LANES)
            segv = segs_v[jj, sl]
            wv = w_v[jj, sl]
            own = (segv >= seg_base) & (segv < seg_base + HALF_SEG)
            wok = jnp.where(own, wv, 0.0)
            segloc = jnp.where(own, segv - seg_base, 0)
            segidx_v[slot, sl] = segloc
            ridx = lax.iota(jnp.int32, NUM_LANES) + (g * NUM_LANES)
            rslot = rows_v.at[slot]
            for d in range(DIM):
                cidx = jnp.full((NUM_LANES,), d, jnp.int32)
                v = plsc.load_gather(rslot, [ridx, cidx])
                plsc.store_scatter(rslot, [ridx, cidx], v * wok)

    for i in range(2):
        pl.when(i < n_my)(lambda i=i: g_desc(i, i).start())

    zero = jnp.zeros((NUM_LANES,), jnp.float32)

    zbuf = rows_v.at[NSLOT - 1]

    @pl.loop(0, BLK)
    def _(i):
        zbuf[i, :] = zero

    row0 = s * ROWS_PER_SUB

    @pl.loop(0, ROWS_PER_SUB // BLK)
    def _(k):
        pltpu.sync_copy(zbuf, acc_sh.at[pl.ds(row0 + k * BLK, BLK)])

    plsc.subcore_barrier()

    @pl.loop(0, (n_my + NSLOT - 1) // NSLOT)
    def _(k):
        for i in range(NSLOT):
            jj = k * NSLOT + i
            b2 = (i + 2) % NSLOT

            @pl.when(jj + 2 < n_my)
            def _():
                pl.when(jj >= 2)(lambda: s_desc(b2).wait())
                g_desc(jj + 2, b2).start()

            @pl.when(jj < n_my)
            def _():
                g_desc(jj, i).wait()
                compute(jj, i)
                s_desc(i).start(add=True)

    for i in range(NSLOT):
        pl.when((n_my >= NSLOT) | (i < n_my))(lambda i=i: s_desc(i).wait())

    plsc.subcore_barrier()

    pltpu.sync_copy(acc_sh.at[pl.ds(row0, ROWS_PER_SUB)],
                    out_hbm.at[pl.ds(seg_base + row0, ROWS_PER_SUB)])


def kernel(feat_ids, feat_weights, segment_ids, embedding):
    ids = feat_ids.astype(jnp.int32)
    segs = segment_ids.astype(jnp.int32)
    w = feat_weights.astype(jnp.float32)

    split = jnp.searchsorted(segs, HALF_SEG).astype(jnp.int32)
    hi0 = -(-(-(-split // BLK)) // 8) * 8
    lo1 = (split // BLK) // 8 * 8
    bounds = jnp.pad(jnp.stack([hi0, lo1]), (0, 14)).astype(jnp.int32)

    pad = PAD_LEN - NNZ
    ids_p = jnp.pad(ids, (0, pad)).reshape(NB_PAD, BLK)
    w_p = jnp.pad(w, (0, pad)).reshape(NB_PAD, BLK)
    segs_p = jnp.pad(segs, (0, pad)).reshape(NB_PAD, BLK)

    pooled = _embed_sum(ids_p, w_p, segs_p, embedding, bounds)
    return pooled.reshape(BATCH, FIELD_COUNT * DIM)
